# baseline (speedup 1.0000x reference)
import jax
import jax.numpy as jnp
from jax.experimental import pallas as pl
from jax.experimental.pallas import tpu as pltpu

TOPK = 5
COST = 0.001
N_ASSETS = 1000
BATCH = 16384
ROW_BLOCK = 1024
N_BLOCKS = BATCH // ROW_BLOCK

_IDX_BITS = 1023


def _rows_kernel(p_ref, t_ref, part_ref):
    p = p_ref[...]
    t = t_ref[...]
    b = jax.lax.bitcast_convert_type(p, jnp.int32)
    col = jax.lax.broadcasted_iota(jnp.int32, p.shape, 1)
    keyed = (b & jnp.int32(~_IDX_BITS)) | (jnp.int32(_IDX_BITS) - col)
    key = jax.lax.bitcast_convert_type(keyed, jnp.float32)

    ninf = jnp.float32(-jnp.inf)
    pinf = jnp.float32(jnp.inf)

    km = key
    for _ in range(TOPK):
        cur = jnp.max(km, axis=1, keepdims=True)
        km = jnp.where(km == cur, ninf, km)
    topmask = km == ninf

    kn = key
    for _ in range(TOPK):
        cur = jnp.min(kn, axis=1, keepdims=True)
        kn = jnp.where(kn == cur, pinf, kn)
    botmask = kn == pinf

    inv_k = jnp.float32(1.0 / TOPK)
    contrib = jnp.where(botmask, -t, jnp.where(topmask, t, 0.0))
    gross = inv_k * jnp.sum(contrib, axis=1)
    overlap = jnp.sum((topmask & botmask).astype(jnp.float32), axis=1)
    ret = gross - COST * inv_k * (2.0 * TOPK - overlap)
    s1 = jnp.sum(ret)
    s2 = jnp.sum(ret * ret)
    part_ref[...] = jnp.concatenate(
        [s1[None, None], s2[None, None]], axis=1)[None]


def _sharpe_kernel(part_ref, o_ref):
    s1 = jnp.sum(part_ref[:, 0, 0])
    s2 = jnp.sum(part_ref[:, 0, 1])
    n = BATCH
    mean = s1 / n
    var = (s2 - n * mean * mean) / (n - 1)
    std = jnp.sqrt(var) + 1e-8
    o_ref[...] = jnp.full((1, 1), -(mean / std), dtype=jnp.float32)


def _stage1(p_loc, t_loc):
    n_blocks = p_loc.shape[0] // ROW_BLOCK
    return pl.pallas_call(
        _rows_kernel,
        grid=(n_blocks,),
        in_specs=[
            pl.BlockSpec((ROW_BLOCK, N_ASSETS), lambda i: (i, 0)),
            pl.BlockSpec((ROW_BLOCK, N_ASSETS), lambda i: (i, 0)),
        ],
        out_specs=pl.BlockSpec((1, 1, 2), lambda i: (i, 0, 0)),
        out_shape=jax.ShapeDtypeStruct((n_blocks, 1, 2), jnp.float32),
        compiler_params=pltpu.CompilerParams(
            dimension_semantics=("parallel",),
        ),
    )(p_loc, t_loc)


def _stage2(parts):
    return pl.pallas_call(
        _sharpe_kernel,
        out_shape=jax.ShapeDtypeStruct((1, 1), jnp.float32),
    )(parts)


@jax.jit
def kernel(predictions, targets):
    devs = jax.devices()
    if len(devs) >= 2:
        mesh = jax.sharding.Mesh(devs[:2], ("x",))
        P = jax.sharding.PartitionSpec
        stage1 = jax.shard_map(
            _stage1, mesh=mesh,
            in_specs=(P("x", None), P("x", None)),
            out_specs=P("x", None, None),
            check_vma=False,
        )
        parts = stage1(predictions, targets)
        parts = jax.lax.with_sharding_constraint(
            parts, jax.sharding.NamedSharding(mesh, P()))
        sharpe = jax.shard_map(
            _stage2, mesh=mesh, in_specs=(P(),), out_specs=P(),
            check_vma=False,
        )
        out = sharpe(parts)
    else:
        out = _stage2(_stage1(predictions, targets))
    return out[0, 0]


# speedup vs baseline: 2.6028x; 2.6028x over previous
import jax
import jax.numpy as jnp
from jax.experimental import pallas as pl
from jax.experimental.pallas import tpu as pltpu

TOPK = 5
COST = 0.001
N_ASSETS = 1000
BATCH = 16384
ROW_BLOCK = 2048
N_BLOCKS = BATCH // ROW_BLOCK

_IDX_BITS = 1023


def _rows_kernel(p_ref, t_ref, part_ref):
    p = p_ref[...]
    t = t_ref[...]
    b = jax.lax.bitcast_convert_type(p, jnp.int32)
    col = jax.lax.broadcasted_iota(jnp.int32, p.shape, 1)
    keyed = (b & jnp.int32(~_IDX_BITS)) | (jnp.int32(_IDX_BITS) - col)
    key = jax.lax.bitcast_convert_type(keyed, jnp.float32)

    ninf = jnp.float32(-jnp.inf)
    pinf = jnp.float32(jnp.inf)

    km = key
    for _ in range(TOPK):
        cur = jnp.max(km, axis=1, keepdims=True)
        km = jnp.where(km == cur, ninf, km)
    topmask = km == ninf

    kn = key
    for _ in range(TOPK):
        cur = jnp.min(kn, axis=1, keepdims=True)
        kn = jnp.where(kn == cur, pinf, kn)
    botmask = kn == pinf

    inv_k = jnp.float32(1.0 / TOPK)
    contrib = jnp.where(botmask, -t, jnp.where(topmask, t, 0.0))
    gross = inv_k * jnp.sum(contrib, axis=1)
    overlap = jnp.sum((topmask & botmask).astype(jnp.float32), axis=1)
    ret = gross - COST * inv_k * (2.0 * TOPK - overlap)
    s1 = jnp.sum(ret)
    s2 = jnp.sum(ret * ret)
    part_ref[...] = jnp.concatenate(
        [s1[None, None], s2[None, None]], axis=1)[None]


def _sharpe_kernel(part_ref, o_ref):
    s1 = jnp.sum(part_ref[:, 0, 0])
    s2 = jnp.sum(part_ref[:, 0, 1])
    n = BATCH
    mean = s1 / n
    var = (s2 - n * mean * mean) / (n - 1)
    std = jnp.sqrt(var) + 1e-8
    o_ref[...] = jnp.full((1, 1), -(mean / std), dtype=jnp.float32)


@jax.jit
def kernel(predictions, targets):
    parts = pl.pallas_call(
        _rows_kernel,
        grid=(N_BLOCKS,),
        in_specs=[
            pl.BlockSpec((ROW_BLOCK, N_ASSETS), lambda i: (i, 0)),
            pl.BlockSpec((ROW_BLOCK, N_ASSETS), lambda i: (i, 0)),
        ],
        out_specs=pl.BlockSpec((1, 1, 2), lambda i: (i, 0, 0)),
        out_shape=jax.ShapeDtypeStruct((N_BLOCKS, 1, 2), jnp.float32),
        compiler_params=pltpu.CompilerParams(
            dimension_semantics=("parallel",),
        ),
    )(predictions, targets)

    out = pl.pallas_call(
        _sharpe_kernel,
        out_shape=jax.ShapeDtypeStruct((1, 1), jnp.float32),
    )(parts)
    return out[0, 0]


# speedup vs baseline: 2.6125x; 1.0037x over previous
import jax
import jax.numpy as jnp
from jax.experimental import pallas as pl
from jax.experimental.pallas import tpu as pltpu

TOPK = 5
COST = 0.001
N_ASSETS = 1000
BATCH = 16384
ROW_BLOCK = 1024
N_BLOCKS = BATCH // ROW_BLOCK

_IDX_BITS = 1023


def _rows_kernel(p_ref, t_ref, part_ref):
    p = p_ref[...]
    t = t_ref[...]
    b = jax.lax.bitcast_convert_type(p, jnp.int32)
    col = jax.lax.broadcasted_iota(jnp.int32, p.shape, 1)
    keyed = (b & jnp.int32(~_IDX_BITS)) | (jnp.int32(_IDX_BITS) - col)
    key = jax.lax.bitcast_convert_type(keyed, jnp.float32)

    ninf = jnp.float32(-jnp.inf)
    pinf = jnp.float32(jnp.inf)

    km = key
    for _ in range(TOPK):
        cur = jnp.max(km, axis=1, keepdims=True)
        km = jnp.where(km == cur, ninf, km)
    topmask = km == ninf

    kn = key
    for _ in range(TOPK):
        cur = jnp.min(kn, axis=1, keepdims=True)
        kn = jnp.where(kn == cur, pinf, kn)
    botmask = kn == pinf

    inv_k = jnp.float32(1.0 / TOPK)
    contrib = jnp.where(botmask, -t, jnp.where(topmask, t, 0.0))
    gross = inv_k * jnp.sum(contrib, axis=1)
    overlap = jnp.sum((topmask & botmask).astype(jnp.float32), axis=1)
    ret = gross - COST * inv_k * (2.0 * TOPK - overlap)
    s1 = jnp.sum(ret)
    s2 = jnp.sum(ret * ret)
    part_ref[...] = jnp.concatenate(
        [s1[None, None], s2[None, None]], axis=1)[None]


def _sharpe_kernel(part_ref, o_ref):
    s1 = jnp.sum(part_ref[:, 0, 0])
    s2 = jnp.sum(part_ref[:, 0, 1])
    n = BATCH
    mean = s1 / n
    var = (s2 - n * mean * mean) / (n - 1)
    std = jnp.sqrt(var) + 1e-8
    o_ref[...] = jnp.full((1, 1), -(mean / std), dtype=jnp.float32)


@jax.jit
def kernel(predictions, targets):
    parts = pl.pallas_call(
        _rows_kernel,
        grid=(N_BLOCKS,),
        in_specs=[
            pl.BlockSpec((ROW_BLOCK, N_ASSETS), lambda i: (i, 0)),
            pl.BlockSpec((ROW_BLOCK, N_ASSETS), lambda i: (i, 0)),
        ],
        out_specs=pl.BlockSpec((1, 1, 2), lambda i: (i, 0, 0)),
        out_shape=jax.ShapeDtypeStruct((N_BLOCKS, 1, 2), jnp.float32),
        compiler_params=pltpu.CompilerParams(
            dimension_semantics=("parallel",),
        ),
    )(predictions, targets)

    out = pl.pallas_call(
        _sharpe_kernel,
        out_shape=jax.ShapeDtypeStruct((1, 1), jnp.float32),
    )(parts)
    return out[0, 0]
